# baseline (speedup 1.0000x reference)
import functools

import jax
import jax.numpy as jnp
from jax import lax
from jax.experimental import pallas as pl
from jax.experimental.pallas import tpu as pltpu
from jax.experimental.pallas import tpu_sc as plsc

N = 10000
E = 320000
F = 128
NC = 2
NS = 16
NW = NC * NS
ACC = 10240
RPT = ACC // NS
CH = 128
TOTCH = 2560
SAFCH = 32
NCH0 = 160
NPH = 5
CPPB = NCH0 // NPH
DCH = 128
EPT = 10240
DNCHUNK = EPT // DCH
DUMP = 10100
BR = 1000

_mesh = plsc.VectorSubcoreMesh(core_axis_name="c", subcore_axis_name="s")


@functools.partial(
    pl.kernel,
    out_type=jax.ShapeDtypeStruct((NC, ACC), jnp.float32),
    mesh=_mesh,
    scratch_types=[
        pltpu.VMEM((DNCHUNK, DCH), jnp.int32),
        pltpu.VMEM((RPT,), jnp.float32),
        pltpu.VMEM_SHARED((ACC,), jnp.float32),
    ],
)
def _deg_kernel(dst_hbm, out_hbm, dst_v, vals, deg_sh):
    cid = lax.axis_index("c")
    sid = lax.axis_index("s")
    wid = cid * NS + sid
    row0 = pl.multiple_of(sid * RPT, 8)

    def zrow(i, carry):
        vals[pl.ds(i * 16, 16)] = jnp.zeros((16,), jnp.float32)
        return carry

    lax.fori_loop(0, RPT // 16, zrow, 0)
    pltpu.sync_copy(vals, deg_sh.at[pl.ds(row0, RPT)])
    pltpu.sync_copy(dst_hbm.at[wid], dst_v)

    def orow(i, carry):
        vals[pl.ds(i * 16, 16)] = jnp.ones((16,), jnp.float32)
        return carry

    lax.fori_loop(0, DCH // 16, orow, 0)
    plsc.subcore_barrier()

    def chunk(j, carry):
        pltpu.sync_copy(vals.at[pl.ds(0, DCH)], deg_sh.at[dst_v.at[j]], add=True)
        return carry

    lax.fori_loop(0, DNCHUNK, chunk, 0)
    plsc.subcore_barrier()
    pltpu.sync_copy(deg_sh.at[pl.ds(row0, RPT)],
                    out_hbm.at[cid, pl.ds(row0, RPT)])


@functools.partial(
    pl.kernel,
    out_type=jax.ShapeDtypeStruct((NC, ACC, F), jnp.float32),
    mesh=_mesh,
    scratch_types=[
        pltpu.VMEM((CPPB, CH), jnp.int32),
        pltpu.VMEM((CPPB, CH), jnp.int32),
        pltpu.VMEM((CH, F), jnp.float32),
        pltpu.VMEM((CH, F), jnp.float32),
        pltpu.VMEM_SHARED((ACC, F), jnp.float32),
        pltpu.SemaphoreType.DMA,
        pltpu.SemaphoreType.DMA,
    ],
)
def _spmv_kernel(x_hbm, src_hbm, dst_hbm, out_hbm,
                 src_v, dst_v, rows_a, rows_b, acc_sh, sem_a, sem_b):
    cid = lax.axis_index("c")
    sid = lax.axis_index("s")
    row0 = pl.multiple_of(sid * RPT, 8)

    def zrow(i, carry):
        for l in range(F // 16):
            rows_a[i, pl.ds(l * 16, 16)] = jnp.zeros((16,), jnp.float32)
        return carry

    lax.fori_loop(0, CH, zrow, 0)

    def zcp(k, carry):
        pltpu.sync_copy(
            rows_a, acc_sh.at[pl.ds(pl.multiple_of(row0 + k * CH, 8), CH)])
        return carry

    lax.fori_loop(0, RPT // CH, zcp, 0)
    plsc.subcore_barrier()

    CPP = NCH0 // NPH

    @pl.when(cid == 0)
    def _edge_work():
        for ph in range(NPH):
            base_ch = sid * NCH0 + ph * CPP
            pltpu.sync_copy(src_hbm.at[pl.ds(base_ch, CPPB)], src_v)
            pltpu.sync_copy(dst_hbm.at[pl.ds(base_ch, CPPB)], dst_v)
            pltpu.async_copy(x_hbm.at[src_v.at[0]], rows_a, sem_a)

            def pair(g, carry):
                j0 = g * 2
                pltpu.async_copy(x_hbm.at[src_v.at[j0 + 1]], rows_b, sem_b)
                pltpu.make_async_copy(x_hbm.at[src_v.at[j0]], rows_a, sem_a).wait()
                pltpu.sync_copy(rows_a, acc_sh.at[dst_v.at[j0]], add=True)
                pltpu.async_copy(x_hbm.at[src_v.at[j0 + 2]], rows_a, sem_a)
                pltpu.make_async_copy(x_hbm.at[src_v.at[j0 + 1]], rows_b, sem_b).wait()
                pltpu.sync_copy(rows_b, acc_sh.at[dst_v.at[j0 + 1]], add=True)
                return carry

            lax.fori_loop(0, CPP // 2 - 1, pair, 0)

            pltpu.async_copy(x_hbm.at[src_v.at[CPP - 1]], rows_b, sem_b)
            pltpu.make_async_copy(x_hbm.at[src_v.at[CPP - 2]], rows_a, sem_a).wait()
            pltpu.sync_copy(rows_a, acc_sh.at[dst_v.at[CPP - 2]], add=True)
            pltpu.make_async_copy(x_hbm.at[src_v.at[CPP - 1]], rows_b, sem_b).wait()
            pltpu.sync_copy(rows_b, acc_sh.at[dst_v.at[CPP - 1]], add=True)

    plsc.subcore_barrier()
    pltpu.sync_copy(acc_sh.at[pl.ds(row0, RPT)],
                    out_hbm.at[cid, pl.ds(row0, RPT)])


def _prescale_body(deg_ref, feat_ref, y_ref):
    d = deg_ref[:, 0:1] + deg_ref[:, 1:2]
    nrm = lax.rsqrt(jnp.maximum(d, 1.0))
    y_ref[...] = feat_ref[...] * nrm


def _mid_body(deg_ref, g_ref, y_ref):
    d = deg_ref[:, 0:1] + deg_ref[:, 1:2]
    n2 = 1.0 / jnp.maximum(d, 1.0)
    g = g_ref[0] + g_ref[1]
    y_ref[...] = g * n2


def _final_body(deg_ref, feat_ref, y2_ref, g2_ref, w_ref, bias_ref, o_ref):
    d = jnp.maximum(deg_ref[:, 0:1] + deg_ref[:, 1:2], 1.0)
    s = jnp.sqrt(d)
    n = lax.rsqrt(d)
    a = feat_ref[...]
    t1 = -(y2_ref[...] * s)
    t2 = (g2_ref[0] + g2_ref[1]) * n
    w0 = w_ref[0]
    w1 = w_ref[1]
    w2 = w_ref[2]
    acc = jnp.dot(a, w0 - w2, preferred_element_type=jnp.float32)
    acc = acc + jnp.dot(t1, w1, preferred_element_type=jnp.float32)
    acc = acc + jnp.dot(t2, 2.0 * w2, preferred_element_type=jnp.float32)
    o_ref[...] = acc + bias_ref[...]


_GRID = N // BR


def _prescale(deg, feat):
    return pl.pallas_call(
        _prescale_body,
        grid=(_GRID,),
        in_specs=[pl.BlockSpec((BR, NC), lambda i: (i, 0)),
                  pl.BlockSpec((BR, F), lambda i: (i, 0))],
        out_specs=pl.BlockSpec((BR, F), lambda i: (i, 0)),
        out_shape=jax.ShapeDtypeStruct((N, F), jnp.float32),
    )(deg, feat)


def _mid(deg, g1):
    return pl.pallas_call(
        _mid_body,
        grid=(_GRID,),
        in_specs=[pl.BlockSpec((BR, NC), lambda i: (i, 0)),
                  pl.BlockSpec((NC, BR, F), lambda i: (0, i, 0))],
        out_specs=pl.BlockSpec((BR, F), lambda i: (i, 0)),
        out_shape=jax.ShapeDtypeStruct((N, F), jnp.float32),
    )(deg, g1)


def _final(deg, feat, y2, g2, W, bias2d):
    return pl.pallas_call(
        _final_body,
        grid=(_GRID,),
        in_specs=[pl.BlockSpec((BR, NC), lambda i: (i, 0)),
                  pl.BlockSpec((BR, F), lambda i: (i, 0)),
                  pl.BlockSpec((BR, F), lambda i: (i, 0)),
                  pl.BlockSpec((NC, BR, F), lambda i: (0, i, 0)),
                  pl.BlockSpec((3, F, F), lambda i: (0, 0, 0)),
                  pl.BlockSpec((1, F), lambda i: (0, 0))],
        out_specs=pl.BlockSpec((BR, F), lambda i: (i, 0)),
        out_shape=jax.ShapeDtypeStruct((N, F), jnp.float32),
    )(deg, feat, y2, g2, W, bias2d)


@jax.jit
def kernel(feat, edge_index, W, bias):
    src = edge_index[0].astype(jnp.int32)
    dst = edge_index[1].astype(jnp.int32)
    npadded = (TOTCH + SAFCH) * CH
    src_p = jnp.concatenate(
        [src, jnp.zeros((npadded - E,), jnp.int32)]).reshape(TOTCH + SAFCH, CH)
    dst_pad = jnp.concatenate([dst, jnp.full((npadded - E,), DUMP, jnp.int32)])
    dst_p = dst_pad.reshape(TOTCH + SAFCH, CH)
    dst_d = dst_pad[:NW * EPT].reshape(NW, DNCHUNK, DCH)

    deg = _deg_kernel(dst_d).T
    y1 = _prescale(deg, feat)
    g1 = _spmv_kernel(y1, src_p, dst_p)
    y2 = _mid(deg, g1)
    g2 = _spmv_kernel(y2, src_p, dst_p)
    return _final(deg, feat, y2, g2, W, bias.reshape(1, F))


# speedup vs baseline: 3.1467x; 3.1467x over previous
import functools

import jax
import jax.numpy as jnp
from jax import lax
from jax.experimental import pallas as pl
from jax.experimental.pallas import tpu as pltpu
from jax.experimental.pallas import tpu_sc as plsc

N = 10000
E = 320000
F = 128
NC = 2
NS = 16
NW = NC * NS
ACC = 10240
RPT = ACC // NS
CH = 128
TOTCH = 2560
SAFCH = 32
NCH0 = 120
NPH = 5
CPPB = NCH0 // NPH
DCH = 128
EPT = 10240
DNCHUNK = EPT // DCH
DUMP = 10100
BR = 1000

_mesh = plsc.VectorSubcoreMesh(core_axis_name="c", subcore_axis_name="s")


@functools.partial(
    pl.kernel,
    out_type=jax.ShapeDtypeStruct((NC, ACC), jnp.float32),
    mesh=_mesh,
    scratch_types=[
        pltpu.VMEM((DNCHUNK, DCH), jnp.int32),
        pltpu.VMEM((RPT,), jnp.float32),
        pltpu.VMEM_SHARED((ACC,), jnp.float32),
    ],
)
def _deg_kernel(dst_hbm, out_hbm, dst_v, vals, deg_sh):
    cid = lax.axis_index("c")
    sid = lax.axis_index("s")
    wid = cid * NS + sid
    row0 = pl.multiple_of(sid * RPT, 8)

    def zrow(i, carry):
        vals[pl.ds(i * 16, 16)] = jnp.zeros((16,), jnp.float32)
        return carry

    lax.fori_loop(0, RPT // 16, zrow, 0)
    pltpu.sync_copy(vals, deg_sh.at[pl.ds(row0, RPT)])
    pltpu.sync_copy(dst_hbm.at[wid], dst_v)

    def orow(i, carry):
        vals[pl.ds(i * 16, 16)] = jnp.ones((16,), jnp.float32)
        return carry

    lax.fori_loop(0, DCH // 16, orow, 0)
    plsc.subcore_barrier()

    def chunk(j, carry):
        pltpu.sync_copy(vals.at[pl.ds(0, DCH)], deg_sh.at[dst_v.at[j]], add=True)
        return carry

    lax.fori_loop(0, DNCHUNK, chunk, 0)
    plsc.subcore_barrier()
    pltpu.sync_copy(deg_sh.at[pl.ds(row0, RPT)],
                    out_hbm.at[cid, pl.ds(row0, RPT)])


@functools.partial(
    pl.kernel,
    out_type=jax.ShapeDtypeStruct((NC, ACC, F), jnp.float32),
    mesh=_mesh,
    scratch_types=[
        pltpu.VMEM((CPPB, CH), jnp.int32),
        pltpu.VMEM((CPPB, CH), jnp.int32),
        pltpu.VMEM((CH, F), jnp.float32),
        pltpu.VMEM((CH, F), jnp.float32),
        pltpu.VMEM_SHARED((ACC, F), jnp.float32),
        pltpu.SemaphoreType.DMA,
        pltpu.SemaphoreType.DMA,
    ],
)
def _spmv_kernel(x_hbm, src_hbm, dst_hbm, out_hbm,
                 src_v, dst_v, rows_a, rows_b, acc_sh, sem_a, sem_b):
    cid = lax.axis_index("c")
    sid = lax.axis_index("s")
    row0 = pl.multiple_of(sid * RPT, 8)

    def zrow(i, carry):
        for l in range(F // 16):
            rows_a[i, pl.ds(l * 16, 16)] = jnp.zeros((16,), jnp.float32)
        return carry

    lax.fori_loop(0, CH, zrow, 0)

    def zcp(k, carry):
        pltpu.sync_copy(
            rows_a, acc_sh.at[pl.ds(pl.multiple_of(row0 + k * CH, 8), CH)])
        return carry

    lax.fori_loop(0, RPT // CH, zcp, 0)
    plsc.subcore_barrier()

    CPP = NCH0 // NPH

    @pl.when(cid == 0)
    def _edge_work():
        for ph in range(NPH):
            base_ch = sid * NCH0 + ph * CPP
            pltpu.sync_copy(src_hbm.at[pl.ds(base_ch, CPPB)], src_v)
            pltpu.sync_copy(dst_hbm.at[pl.ds(base_ch, CPPB)], dst_v)
            pltpu.async_copy(x_hbm.at[src_v.at[0]], rows_a, sem_a)

            def pair(g, carry):
                j0 = g * 2
                pltpu.async_copy(x_hbm.at[src_v.at[j0 + 1]], rows_b, sem_b)
                pltpu.make_async_copy(x_hbm.at[src_v.at[j0]], rows_a, sem_a).wait()
                pltpu.sync_copy(rows_a, acc_sh.at[dst_v.at[j0]], add=True)
                pltpu.async_copy(x_hbm.at[src_v.at[j0 + 2]], rows_a, sem_a)
                pltpu.make_async_copy(x_hbm.at[src_v.at[j0 + 1]], rows_b, sem_b).wait()
                pltpu.sync_copy(rows_b, acc_sh.at[dst_v.at[j0 + 1]], add=True)
                return carry

            lax.fori_loop(0, CPP // 2 - 1, pair, 0)

            pltpu.async_copy(x_hbm.at[src_v.at[CPP - 1]], rows_b, sem_b)
            pltpu.make_async_copy(x_hbm.at[src_v.at[CPP - 2]], rows_a, sem_a).wait()
            pltpu.sync_copy(rows_a, acc_sh.at[dst_v.at[CPP - 2]], add=True)
            pltpu.make_async_copy(x_hbm.at[src_v.at[CPP - 1]], rows_b, sem_b).wait()
            pltpu.sync_copy(rows_b, acc_sh.at[dst_v.at[CPP - 1]], add=True)

    plsc.subcore_barrier()
    pltpu.sync_copy(acc_sh.at[pl.ds(row0, RPT)],
                    out_hbm.at[cid, pl.ds(row0, RPT)])


def _prescale_body(deg_ref, feat_ref, y_ref):
    d = deg_ref[:, 0:1] + deg_ref[:, 1:2]
    nrm = lax.rsqrt(jnp.maximum(d, 1.0))
    y_ref[...] = feat_ref[...] * nrm


def _mid_body(deg_ref, g_ref, y_ref):
    d = deg_ref[:, 0:1] + deg_ref[:, 1:2]
    n2 = 1.0 / jnp.maximum(d, 1.0)
    g = g_ref[0] + g_ref[1]
    y_ref[...] = g * n2


def _final_body(deg_ref, feat_ref, y2_ref, g2_ref, w_ref, bias_ref, o_ref):
    d = jnp.maximum(deg_ref[:, 0:1] + deg_ref[:, 1:2], 1.0)
    s = jnp.sqrt(d)
    n = lax.rsqrt(d)
    a = feat_ref[...]
    t1 = -(y2_ref[...] * s)
    t2 = (g2_ref[0] + g2_ref[1]) * n
    w0 = w_ref[0]
    w1 = w_ref[1]
    w2 = w_ref[2]
    acc = jnp.dot(a, w0 - w2, preferred_element_type=jnp.float32)
    acc = acc + jnp.dot(t1, w1, preferred_element_type=jnp.float32)
    acc = acc + jnp.dot(t2, 2.0 * w2, preferred_element_type=jnp.float32)
    o_ref[...] = acc + bias_ref[...]


_GRID = N // BR


def _prescale(deg, feat):
    return pl.pallas_call(
        _prescale_body,
        grid=(_GRID,),
        in_specs=[pl.BlockSpec((BR, NC), lambda i: (i, 0)),
                  pl.BlockSpec((BR, F), lambda i: (i, 0))],
        out_specs=pl.BlockSpec((BR, F), lambda i: (i, 0)),
        out_shape=jax.ShapeDtypeStruct((N, F), jnp.float32),
    )(deg, feat)


def _mid(deg, g1):
    return pl.pallas_call(
        _mid_body,
        grid=(_GRID,),
        in_specs=[pl.BlockSpec((BR, NC), lambda i: (i, 0)),
                  pl.BlockSpec((NC, BR, F), lambda i: (0, i, 0))],
        out_specs=pl.BlockSpec((BR, F), lambda i: (i, 0)),
        out_shape=jax.ShapeDtypeStruct((N, F), jnp.float32),
    )(deg, g1)


def _final(deg, feat, y2, g2, W, bias2d):
    return pl.pallas_call(
        _final_body,
        grid=(_GRID,),
        in_specs=[pl.BlockSpec((BR, NC), lambda i: (i, 0)),
                  pl.BlockSpec((BR, F), lambda i: (i, 0)),
                  pl.BlockSpec((BR, F), lambda i: (i, 0)),
                  pl.BlockSpec((NC, BR, F), lambda i: (0, i, 0)),
                  pl.BlockSpec((3, F, F), lambda i: (0, 0, 0)),
                  pl.BlockSpec((1, F), lambda i: (0, 0))],
        out_specs=pl.BlockSpec((BR, F), lambda i: (i, 0)),
        out_shape=jax.ShapeDtypeStruct((N, F), jnp.float32),
    )(deg, feat, y2, g2, W, bias2d)


@jax.jit
def kernel(feat, edge_index, W, bias):
    src = edge_index[0].astype(jnp.int32)
    dst = edge_index[1].astype(jnp.int32)
    npadded = (TOTCH + SAFCH) * CH
    src_p = jnp.concatenate(
        [src, jnp.zeros((npadded - E,), jnp.int32)]).reshape(TOTCH + SAFCH, CH)
    dst_pad = jnp.concatenate([dst, jnp.full((npadded - E,), DUMP, jnp.int32)])
    dst_p = dst_pad.reshape(TOTCH + SAFCH, CH)
    dst_d = dst_pad[:NW * EPT].reshape(NW, DNCHUNK, DCH)

    deg = _deg_kernel(dst_d).T
    y1 = _prescale(deg, feat)
    g1 = _spmv_kernel(y1, src_p, dst_p)
    y2 = _mid(deg, g1)
    g2 = _spmv_kernel(y2, src_p, dst_p)
    return _final(deg, feat, y2, g2, W, bias.reshape(1, F))


# speedup vs baseline: 4.2256x; 1.3429x over previous
import functools

import jax
import jax.numpy as jnp
from jax import lax
from jax.experimental import pallas as pl
from jax.experimental.pallas import tpu as pltpu
from jax.experimental.pallas import tpu_sc as plsc

N = 10000
E = 320000
F = 128
NC = 2
NS = 16
NW = NC * NS
ACC = 10240
RPT = ACC // NS
CH = 128
TOTCH = 2560
SAFCH = 32
NCH = 80
NPH = 2
CPPB = NCH // NPH
DCH = 128
EPT = 10240
DNCHUNK = EPT // DCH
DUMP = 10100
BR = 1000

_mesh = plsc.VectorSubcoreMesh(core_axis_name="c", subcore_axis_name="s")


@functools.partial(
    pl.kernel,
    out_type=jax.ShapeDtypeStruct((NC, ACC), jnp.float32),
    mesh=_mesh,
    scratch_types=[
        pltpu.VMEM((DNCHUNK, DCH), jnp.int32),
        pltpu.VMEM((RPT,), jnp.float32),
        pltpu.VMEM_SHARED((ACC,), jnp.float32),
    ],
)
def _deg_kernel(dst_hbm, out_hbm, dst_v, vals, deg_sh):
    cid = lax.axis_index("c")
    sid = lax.axis_index("s")
    wid = cid * NS + sid
    row0 = pl.multiple_of(sid * RPT, 8)

    def zrow(i, carry):
        vals[pl.ds(i * 16, 16)] = jnp.zeros((16,), jnp.float32)
        return carry

    lax.fori_loop(0, RPT // 16, zrow, 0)
    pltpu.sync_copy(vals, deg_sh.at[pl.ds(row0, RPT)])
    pltpu.sync_copy(dst_hbm.at[wid], dst_v)

    def orow(i, carry):
        vals[pl.ds(i * 16, 16)] = jnp.ones((16,), jnp.float32)
        return carry

    lax.fori_loop(0, DCH // 16, orow, 0)
    plsc.subcore_barrier()

    def chunk(j, carry):
        pltpu.sync_copy(vals.at[pl.ds(0, DCH)], deg_sh.at[dst_v.at[j]], add=True)
        return carry

    lax.fori_loop(0, DNCHUNK, chunk, 0)
    plsc.subcore_barrier()
    pltpu.sync_copy(deg_sh.at[pl.ds(row0, RPT)],
                    out_hbm.at[cid, pl.ds(row0, RPT)])


@functools.partial(
    pl.kernel,
    out_type=jax.ShapeDtypeStruct((NC, ACC, F), jnp.float32),
    mesh=_mesh,
    scratch_types=[
        pltpu.VMEM((CPPB, CH), jnp.int32),
        pltpu.VMEM((CPPB, CH), jnp.int32),
        pltpu.VMEM((CH, F), jnp.float32),
        pltpu.VMEM((CH, F), jnp.float32),
        pltpu.VMEM_SHARED((ACC, F), jnp.float32),
        pltpu.SemaphoreType.DMA,
        pltpu.SemaphoreType.DMA,
    ],
)
def _spmv_kernel(x_hbm, src_hbm, dst_hbm, out_hbm,
                 src_v, dst_v, rows_a, rows_b, acc_sh, sem_a, sem_b):
    cid = lax.axis_index("c")
    sid = lax.axis_index("s")
    row0 = pl.multiple_of(sid * RPT, 8)

    def zrow(i, carry):
        for l in range(F // 16):
            rows_a[i, pl.ds(l * 16, 16)] = jnp.zeros((16,), jnp.float32)
        return carry

    lax.fori_loop(0, CH, zrow, 0)

    def zcp(k, carry):
        pltpu.sync_copy(
            rows_a, acc_sh.at[pl.ds(pl.multiple_of(row0 + k * CH, 8), CH)])
        return carry

    lax.fori_loop(0, RPT // CH, zcp, 0)
    plsc.subcore_barrier()

    CPP = NCH // NPH
    wid = cid * NS + sid

    for ph in range(NPH):
        base_ch = wid * NCH + ph * CPP
        pltpu.sync_copy(src_hbm.at[pl.ds(base_ch, CPPB)], src_v)
        pltpu.sync_copy(dst_hbm.at[pl.ds(base_ch, CPPB)], dst_v)
        pltpu.async_copy(x_hbm.at[src_v.at[0]], rows_a, sem_a)

        def pair(g, carry):
            j0 = g * 2
            pltpu.async_copy(x_hbm.at[src_v.at[j0 + 1]], rows_b, sem_b)
            pltpu.make_async_copy(x_hbm.at[src_v.at[j0]], rows_a, sem_a).wait()
            pltpu.sync_copy(rows_a, acc_sh.at[dst_v.at[j0]], add=True)
            pltpu.async_copy(x_hbm.at[src_v.at[j0 + 2]], rows_a, sem_a)
            pltpu.make_async_copy(x_hbm.at[src_v.at[j0 + 1]], rows_b, sem_b).wait()
            pltpu.sync_copy(rows_b, acc_sh.at[dst_v.at[j0 + 1]], add=True)
            return carry

        lax.fori_loop(0, CPP // 2 - 1, pair, 0)

        pltpu.async_copy(x_hbm.at[src_v.at[CPP - 1]], rows_b, sem_b)
        pltpu.make_async_copy(x_hbm.at[src_v.at[CPP - 2]], rows_a, sem_a).wait()
        pltpu.sync_copy(rows_a, acc_sh.at[dst_v.at[CPP - 2]], add=True)
        pltpu.make_async_copy(x_hbm.at[src_v.at[CPP - 1]], rows_b, sem_b).wait()
        pltpu.sync_copy(rows_b, acc_sh.at[dst_v.at[CPP - 1]], add=True)

    plsc.subcore_barrier()
    pltpu.sync_copy(acc_sh.at[pl.ds(row0, RPT)],
                    out_hbm.at[cid, pl.ds(row0, RPT)])


def _prescale_body(deg_ref, feat_ref, y_ref):
    d = deg_ref[:, 0:1] + deg_ref[:, 1:2]
    nrm = lax.rsqrt(jnp.maximum(d, 1.0))
    y_ref[...] = feat_ref[...] * nrm


def _mid_body(deg_ref, g_ref, y_ref):
    d = deg_ref[:, 0:1] + deg_ref[:, 1:2]
    n2 = 1.0 / jnp.maximum(d, 1.0)
    g = g_ref[0] + g_ref[1]
    y_ref[...] = g * n2


def _final_body(deg_ref, feat_ref, y2_ref, g2_ref, w_ref, bias_ref, o_ref):
    d = jnp.maximum(deg_ref[:, 0:1] + deg_ref[:, 1:2], 1.0)
    s = jnp.sqrt(d)
    n = lax.rsqrt(d)
    a = feat_ref[...]
    t1 = -(y2_ref[...] * s)
    t2 = (g2_ref[0] + g2_ref[1]) * n
    w0 = w_ref[0]
    w1 = w_ref[1]
    w2 = w_ref[2]
    acc = jnp.dot(a, w0 - w2, preferred_element_type=jnp.float32)
    acc = acc + jnp.dot(t1, w1, preferred_element_type=jnp.float32)
    acc = acc + jnp.dot(t2, 2.0 * w2, preferred_element_type=jnp.float32)
    o_ref[...] = acc + bias_ref[...]


_GRID = N // BR


def _prescale(deg, feat):
    return pl.pallas_call(
        _prescale_body,
        grid=(_GRID,),
        in_specs=[pl.BlockSpec((BR, NC), lambda i: (i, 0)),
                  pl.BlockSpec((BR, F), lambda i: (i, 0))],
        out_specs=pl.BlockSpec((BR, F), lambda i: (i, 0)),
        out_shape=jax.ShapeDtypeStruct((N, F), jnp.float32),
    )(deg, feat)


def _mid(deg, g1):
    return pl.pallas_call(
        _mid_body,
        grid=(_GRID,),
        in_specs=[pl.BlockSpec((BR, NC), lambda i: (i, 0)),
                  pl.BlockSpec((NC, BR, F), lambda i: (0, i, 0))],
        out_specs=pl.BlockSpec((BR, F), lambda i: (i, 0)),
        out_shape=jax.ShapeDtypeStruct((N, F), jnp.float32),
    )(deg, g1)


def _final(deg, feat, y2, g2, W, bias2d):
    return pl.pallas_call(
        _final_body,
        grid=(_GRID,),
        in_specs=[pl.BlockSpec((BR, NC), lambda i: (i, 0)),
                  pl.BlockSpec((BR, F), lambda i: (i, 0)),
                  pl.BlockSpec((BR, F), lambda i: (i, 0)),
                  pl.BlockSpec((NC, BR, F), lambda i: (0, i, 0)),
                  pl.BlockSpec((3, F, F), lambda i: (0, 0, 0)),
                  pl.BlockSpec((1, F), lambda i: (0, 0))],
        out_specs=pl.BlockSpec((BR, F), lambda i: (i, 0)),
        out_shape=jax.ShapeDtypeStruct((N, F), jnp.float32),
    )(deg, feat, y2, g2, W, bias2d)


@jax.jit
def kernel(feat, edge_index, W, bias):
    src = edge_index[0].astype(jnp.int32)
    dst = edge_index[1].astype(jnp.int32)
    npadded = (TOTCH + SAFCH) * CH
    npad = npadded - E
    pad_iota = jnp.arange(npad, dtype=jnp.int32)
    src_p = jnp.concatenate([src, pad_iota % N]).reshape(TOTCH + SAFCH, CH)
    dst_pad = jnp.concatenate([dst, N + pad_iota % (ACC - N)])
    dst_p = dst_pad.reshape(TOTCH + SAFCH, CH)
    dst_d = dst_pad[:NW * EPT].reshape(NW, DNCHUNK, DCH)

    deg = _deg_kernel(dst_d).T
    y1 = _prescale(deg, feat)
    g1 = _spmv_kernel(y1, src_p, dst_p)
    y2 = _mid(deg, g1)
    g2 = _spmv_kernel(y2, src_p, dst_p)
    return _final(deg, feat, y2, g2, W, bias.reshape(1, F))


# speedup vs baseline: 4.3055x; 1.0189x over previous
import functools

import jax
import jax.numpy as jnp
from jax import lax
from jax.experimental import pallas as pl
from jax.experimental.pallas import tpu as pltpu
from jax.experimental.pallas import tpu_sc as plsc

N = 10000
E = 320000
F = 128
NC = 2
NS = 16
NW = NC * NS
ACC = 10240
RPT = ACC // NS
CH = 64
TOTCH = 5120
SAFCH = 32
NCH = 160
NPH = 5
CPPB = NCH // NPH
DCH = 128
EPT = 10240
DNCHUNK = EPT // DCH
DUMP = 10100
BR = 1000

_mesh = plsc.VectorSubcoreMesh(core_axis_name="c", subcore_axis_name="s")


@functools.partial(
    pl.kernel,
    out_type=jax.ShapeDtypeStruct((NC, ACC), jnp.float32),
    mesh=_mesh,
    scratch_types=[
        pltpu.VMEM((DNCHUNK, DCH), jnp.int32),
        pltpu.VMEM((RPT,), jnp.float32),
        pltpu.VMEM_SHARED((ACC,), jnp.float32),
    ],
)
def _deg_kernel(dst_hbm, out_hbm, dst_v, vals, deg_sh):
    cid = lax.axis_index("c")
    sid = lax.axis_index("s")
    wid = cid * NS + sid
    row0 = pl.multiple_of(sid * RPT, 8)

    def zrow(i, carry):
        vals[pl.ds(i * 16, 16)] = jnp.zeros((16,), jnp.float32)
        return carry

    lax.fori_loop(0, RPT // 16, zrow, 0)
    pltpu.sync_copy(vals, deg_sh.at[pl.ds(row0, RPT)])
    pltpu.sync_copy(dst_hbm.at[wid], dst_v)

    def orow(i, carry):
        vals[pl.ds(i * 16, 16)] = jnp.ones((16,), jnp.float32)
        return carry

    lax.fori_loop(0, DCH // 16, orow, 0)
    plsc.subcore_barrier()

    def chunk(j, carry):
        pltpu.sync_copy(vals.at[pl.ds(0, DCH)], deg_sh.at[dst_v.at[j]], add=True)
        return carry

    lax.fori_loop(0, DNCHUNK, chunk, 0)
    plsc.subcore_barrier()
    pltpu.sync_copy(deg_sh.at[pl.ds(row0, RPT)],
                    out_hbm.at[cid, pl.ds(row0, RPT)])


@functools.partial(
    pl.kernel,
    out_type=jax.ShapeDtypeStruct((NC, ACC, F), jnp.float32),
    mesh=_mesh,
    scratch_types=[
        pltpu.VMEM((CPPB, CH), jnp.int32),
        pltpu.VMEM((CPPB, CH), jnp.int32),
        pltpu.VMEM((CH, F), jnp.float32),
        pltpu.VMEM((CH, F), jnp.float32),
        pltpu.VMEM((CH, F), jnp.float32),
        pltpu.VMEM((CH, F), jnp.float32),
        pltpu.VMEM_SHARED((ACC, F), jnp.float32),
        pltpu.SemaphoreType.DMA,
        pltpu.SemaphoreType.DMA,
        pltpu.SemaphoreType.DMA,
        pltpu.SemaphoreType.DMA,
    ],
)
def _spmv_kernel(x_hbm, src_hbm, dst_hbm, out_hbm,
                 src_v, dst_v, rows_a, rows_b, rows_c, rows_d, acc_sh,
                 sem_a, sem_b, sem_c, sem_d):
    cid = lax.axis_index("c")
    sid = lax.axis_index("s")
    row0 = pl.multiple_of(sid * RPT, 8)

    def zrow(i, carry):
        for l in range(F // 16):
            rows_a[i, pl.ds(l * 16, 16)] = jnp.zeros((16,), jnp.float32)
        return carry

    lax.fori_loop(0, CH, zrow, 0)

    def zcp(k, carry):
        pltpu.sync_copy(
            rows_a, acc_sh.at[pl.ds(pl.multiple_of(row0 + k * CH, 8), CH)])
        return carry

    lax.fori_loop(0, RPT // CH, zcp, 0)
    bufs = [(rows_a, sem_a), (rows_b, sem_b), (rows_c, sem_c), (rows_d, sem_d)]
    plsc.subcore_barrier()

    CPP = NCH // NPH
    wid = cid * NS + sid

    for ph in range(NPH):
        base_ch = wid * NCH + ph * CPP
        pltpu.sync_copy(src_hbm.at[pl.ds(base_ch, CPPB)], src_v)
        pltpu.sync_copy(dst_hbm.at[pl.ds(base_ch, CPPB)], dst_v)
        for k in range(3):
            pltpu.async_copy(x_hbm.at[src_v.at[k]], bufs[k][0], bufs[k][1])

        def quad(g, carry):
            j0 = g * 4
            for k in range(4):
                rows_k, sem_k = bufs[k]
                rows_n, sem_n = bufs[(k + 3) % 4]

                @pl.when(j0 + k + 3 < CPP)
                def _start():
                    pltpu.async_copy(x_hbm.at[src_v.at[j0 + k + 3]],
                                     rows_n, sem_n)

                pltpu.make_async_copy(x_hbm.at[src_v.at[j0 + k]],
                                      rows_k, sem_k).wait()
                pltpu.sync_copy(rows_k, acc_sh.at[dst_v.at[j0 + k]], add=True)
            return carry

        lax.fori_loop(0, CPP // 4, quad, 0)

    plsc.subcore_barrier()
    pltpu.sync_copy(acc_sh.at[pl.ds(row0, RPT)],
                    out_hbm.at[cid, pl.ds(row0, RPT)])


def _prescale_body(deg_ref, feat_ref, y_ref):
    d = deg_ref[:, 0:1] + deg_ref[:, 1:2]
    nrm = lax.rsqrt(jnp.maximum(d, 1.0))
    y_ref[...] = feat_ref[...] * nrm


def _mid_body(deg_ref, g_ref, y_ref):
    d = deg_ref[:, 0:1] + deg_ref[:, 1:2]
    n2 = 1.0 / jnp.maximum(d, 1.0)
    g = g_ref[0] + g_ref[1]
    y_ref[...] = g * n2


def _final_body(deg_ref, feat_ref, y2_ref, g2_ref, w_ref, bias_ref, o_ref):
    d = jnp.maximum(deg_ref[:, 0:1] + deg_ref[:, 1:2], 1.0)
    s = jnp.sqrt(d)
    n = lax.rsqrt(d)
    a = feat_ref[...]
    t1 = -(y2_ref[...] * s)
    t2 = (g2_ref[0] + g2_ref[1]) * n
    w0 = w_ref[0]
    w1 = w_ref[1]
    w2 = w_ref[2]
    acc = jnp.dot(a, w0 - w2, preferred_element_type=jnp.float32)
    acc = acc + jnp.dot(t1, w1, preferred_element_type=jnp.float32)
    acc = acc + jnp.dot(t2, 2.0 * w2, preferred_element_type=jnp.float32)
    o_ref[...] = acc + bias_ref[...]


_GRID = N // BR


def _prescale(deg, feat):
    return pl.pallas_call(
        _prescale_body,
        grid=(_GRID,),
        in_specs=[pl.BlockSpec((BR, NC), lambda i: (i, 0)),
                  pl.BlockSpec((BR, F), lambda i: (i, 0))],
        out_specs=pl.BlockSpec((BR, F), lambda i: (i, 0)),
        out_shape=jax.ShapeDtypeStruct((N, F), jnp.float32),
    )(deg, feat)


def _mid(deg, g1):
    return pl.pallas_call(
        _mid_body,
        grid=(_GRID,),
        in_specs=[pl.BlockSpec((BR, NC), lambda i: (i, 0)),
                  pl.BlockSpec((NC, BR, F), lambda i: (0, i, 0))],
        out_specs=pl.BlockSpec((BR, F), lambda i: (i, 0)),
        out_shape=jax.ShapeDtypeStruct((N, F), jnp.float32),
    )(deg, g1)


def _final(deg, feat, y2, g2, W, bias2d):
    return pl.pallas_call(
        _final_body,
        grid=(_GRID,),
        in_specs=[pl.BlockSpec((BR, NC), lambda i: (i, 0)),
                  pl.BlockSpec((BR, F), lambda i: (i, 0)),
                  pl.BlockSpec((BR, F), lambda i: (i, 0)),
                  pl.BlockSpec((NC, BR, F), lambda i: (0, i, 0)),
                  pl.BlockSpec((3, F, F), lambda i: (0, 0, 0)),
                  pl.BlockSpec((1, F), lambda i: (0, 0))],
        out_specs=pl.BlockSpec((BR, F), lambda i: (i, 0)),
        out_shape=jax.ShapeDtypeStruct((N, F), jnp.float32),
    )(deg, feat, y2, g2, W, bias2d)


@jax.jit
def kernel(feat, edge_index, W, bias):
    src = edge_index[0].astype(jnp.int32)
    dst = edge_index[1].astype(jnp.int32)
    npadded = (TOTCH + SAFCH) * CH
    npad = npadded - E
    pad_iota = jnp.arange(npad, dtype=jnp.int32)
    src_p = jnp.concatenate([src, pad_iota % N]).reshape(TOTCH + SAFCH, CH)
    dst_pad = jnp.concatenate([dst, N + pad_iota % (ACC - N)])
    dst_p = dst_pad.reshape(TOTCH + SAFCH, CH)
    dst_d = dst_pad[:NW * EPT].reshape(NW, DNCHUNK, DCH)

    deg = _deg_kernel(dst_d).T
    y1 = _prescale(deg, feat)
    g1 = _spmv_kernel(y1, src_p, dst_p)
    y2 = _mid(deg, g1)
    g2 = _spmv_kernel(y2, src_p, dst_p)
    return _final(deg, feat, y2, g2, W, bias.reshape(1, F))


# speedup vs baseline: 4.3078x; 1.0005x over previous
import functools

import jax
import jax.numpy as jnp
from jax import lax
from jax.experimental import pallas as pl
from jax.experimental.pallas import tpu as pltpu
from jax.experimental.pallas import tpu_sc as plsc

N = 10000
E = 320000
F = 128
NC = 2
NS = 16
NW = NC * NS
ACC = 10240
RPT = ACC // NS
CH = 64
TOTCH = 5120
SAFCH = 32
NCH = 160
NPH = 5
CPPB = NCH // NPH
DCH = 128
EPT = 10240
DNCHUNK = EPT // DCH
BR = 1000

_mesh = plsc.VectorSubcoreMesh(core_axis_name="c", subcore_axis_name="s",
                               num_cores=NC, num_subcores=NS)


@functools.partial(
    pl.kernel,
    out_type=jax.ShapeDtypeStruct((NC, ACC), jnp.float32),
    mesh=_mesh,
    scratch_types=[
        pltpu.VMEM((DNCHUNK, DCH), jnp.int32),
        pltpu.VMEM((RPT,), jnp.float32),
        pltpu.VMEM_SHARED((ACC,), jnp.float32),
    ],
)
def _deg_kernel(dst_hbm, out_hbm, dst_v, vals, deg_sh):
    cid = lax.axis_index("c")
    sid = lax.axis_index("s")
    wid = cid * NS + sid
    row0 = pl.multiple_of(sid * RPT, 8)

    def zrow(i, carry):
        vals[pl.ds(i * 16, 16)] = jnp.zeros((16,), jnp.float32)
        return carry

    lax.fori_loop(0, RPT // 16, zrow, 0)
    pltpu.sync_copy(vals, deg_sh.at[pl.ds(row0, RPT)])
    pltpu.sync_copy(dst_hbm.at[wid], dst_v)

    def orow(i, carry):
        vals[pl.ds(i * 16, 16)] = jnp.ones((16,), jnp.float32)
        return carry

    lax.fori_loop(0, DCH // 16, orow, 0)
    plsc.subcore_barrier()

    def chunk(j, carry):
        pltpu.sync_copy(vals.at[pl.ds(0, DCH)], deg_sh.at[dst_v.at[j]], add=True)
        return carry

    lax.fori_loop(0, DNCHUNK, chunk, 0)
    plsc.subcore_barrier()
    pltpu.sync_copy(deg_sh.at[pl.ds(row0, RPT)],
                    out_hbm.at[cid, pl.ds(row0, RPT)])


@functools.partial(
    pl.kernel,
    out_type=jax.ShapeDtypeStruct((NC, ACC, F), jnp.float32),
    mesh=_mesh,
    scratch_types=[
        pltpu.VMEM((CPPB, CH), jnp.int32),
        pltpu.VMEM((CPPB, CH), jnp.int32),
        pltpu.VMEM((CH, F), jnp.float32),
        pltpu.VMEM((CH, F), jnp.float32),
        pltpu.VMEM((CH, F), jnp.float32),
        pltpu.VMEM((CH, F), jnp.float32),
        pltpu.VMEM_SHARED((ACC, F), jnp.float32),
        pltpu.SemaphoreType.DMA,
        pltpu.SemaphoreType.DMA,
        pltpu.SemaphoreType.DMA,
        pltpu.SemaphoreType.DMA,
    ],
)
def _spmv_kernel(x_hbm, src_hbm, dst_hbm, out_hbm,
                 src_v, dst_v, rows_a, rows_b, rows_c, rows_d, acc_sh,
                 sem_a, sem_b, sem_c, sem_d):
    cid = lax.axis_index("c")
    sid = lax.axis_index("s")
    row0 = pl.multiple_of(sid * RPT, 8)

    def zrow(i, carry):
        for l in range(F // 16):
            rows_a[i, pl.ds(l * 16, 16)] = jnp.zeros((16,), jnp.float32)
        return carry

    lax.fori_loop(0, CH, zrow, 0)

    def zcp(k, carry):
        pltpu.sync_copy(
            rows_a, acc_sh.at[pl.ds(pl.multiple_of(row0 + k * CH, 8), CH)])
        return carry

    lax.fori_loop(0, RPT // CH, zcp, 0)
    bufs = [(rows_a, sem_a), (rows_b, sem_b), (rows_c, sem_c), (rows_d, sem_d)]
    plsc.subcore_barrier()

    CPP = NCH // NPH
    wid = cid * NS + sid

    for ph in range(NPH):
        base_ch = wid * NCH + ph * CPP
        pltpu.sync_copy(src_hbm.at[pl.ds(base_ch, CPPB)], src_v)
        pltpu.sync_copy(dst_hbm.at[pl.ds(base_ch, CPPB)], dst_v)
        for k in range(3):
            pltpu.async_copy(x_hbm.at[src_v.at[k]], bufs[k][0], bufs[k][1])

        def quad(g, carry):
            j0 = g * 4
            for k in range(4):
                rows_k, sem_k = bufs[k]
                rows_n, sem_n = bufs[(k + 3) % 4]

                @pl.when(j0 + k + 3 < CPP)
                def _start():
                    pltpu.async_copy(x_hbm.at[src_v.at[j0 + k + 3]],
                                     rows_n, sem_n)

                pltpu.make_async_copy(x_hbm.at[src_v.at[j0 + k]],
                                      rows_k, sem_k).wait()
                pltpu.sync_copy(rows_k, acc_sh.at[dst_v.at[j0 + k]], add=True)
            return carry

        lax.fori_loop(0, CPP // 4, quad, 0)

    plsc.subcore_barrier()
    pltpu.sync_copy(acc_sh.at[pl.ds(row0, RPT)],
                    out_hbm.at[cid, pl.ds(row0, RPT)])


def _prescale_body(deg_ref, feat_ref, y_ref):
    d = deg_ref[:, 0:1] + deg_ref[:, 1:2]
    nrm = lax.rsqrt(jnp.maximum(d, 1.0))
    y_ref[...] = feat_ref[...] * nrm


def _mid_body(deg_ref, g_ref, y_ref):
    d = deg_ref[:, 0:1] + deg_ref[:, 1:2]
    n2 = 1.0 / jnp.maximum(d, 1.0)
    g = g_ref[0] + g_ref[1]
    y_ref[...] = g * n2


def _final_body(deg_ref, feat_ref, y2_ref, g2_ref, w_ref, bias_ref, o_ref):
    d = jnp.maximum(deg_ref[:, 0:1] + deg_ref[:, 1:2], 1.0)
    s = jnp.sqrt(d)
    n = lax.rsqrt(d)
    a = feat_ref[...]
    t1 = -(y2_ref[...] * s)
    t2 = (g2_ref[0] + g2_ref[1]) * n
    w0 = w_ref[0]
    w1 = w_ref[1]
    w2 = w_ref[2]
    acc = jnp.dot(a, w0 - w2, preferred_element_type=jnp.float32)
    acc = acc + jnp.dot(t1, w1, preferred_element_type=jnp.float32)
    acc = acc + jnp.dot(t2, 2.0 * w2, preferred_element_type=jnp.float32)
    o_ref[...] = acc + bias_ref[...]


_GRID = N // BR


def _prescale(deg, feat):
    return pl.pallas_call(
        _prescale_body,
        grid=(_GRID,),
        in_specs=[pl.BlockSpec((BR, NC), lambda i: (i, 0)),
                  pl.BlockSpec((BR, F), lambda i: (i, 0))],
        out_specs=pl.BlockSpec((BR, F), lambda i: (i, 0)),
        out_shape=jax.ShapeDtypeStruct((N, F), jnp.float32),
    )(deg, feat)


def _mid(deg, g1):
    return pl.pallas_call(
        _mid_body,
        grid=(_GRID,),
        in_specs=[pl.BlockSpec((BR, NC), lambda i: (i, 0)),
                  pl.BlockSpec((NC, BR, F), lambda i: (0, i, 0))],
        out_specs=pl.BlockSpec((BR, F), lambda i: (i, 0)),
        out_shape=jax.ShapeDtypeStruct((N, F), jnp.float32),
    )(deg, g1)


def _final(deg, feat, y2, g2, W, bias2d):
    return pl.pallas_call(
        _final_body,
        grid=(_GRID,),
        in_specs=[pl.BlockSpec((BR, NC), lambda i: (i, 0)),
                  pl.BlockSpec((BR, F), lambda i: (i, 0)),
                  pl.BlockSpec((BR, F), lambda i: (i, 0)),
                  pl.BlockSpec((NC, BR, F), lambda i: (0, i, 0)),
                  pl.BlockSpec((3, F, F), lambda i: (0, 0, 0)),
                  pl.BlockSpec((1, F), lambda i: (0, 0))],
        out_specs=pl.BlockSpec((BR, F), lambda i: (i, 0)),
        out_shape=jax.ShapeDtypeStruct((N, F), jnp.float32),
    )(deg, feat, y2, g2, W, bias2d)


@jax.jit
def kernel(feat, edge_index, W, bias):
    src = edge_index[0].astype(jnp.int32)
    dst = edge_index[1].astype(jnp.int32)
    npadded = (TOTCH + SAFCH) * CH
    npad = npadded - E
    pad_iota = jnp.arange(npad, dtype=jnp.int32)
    src_p = jnp.concatenate([src, pad_iota % N]).reshape(TOTCH + SAFCH, CH)
    dst_pad = jnp.concatenate([dst, N + pad_iota % (ACC - N)])
    dst_p = dst_pad.reshape(TOTCH + SAFCH, CH)
    dst_d = dst_pad[:NW * EPT].reshape(NW, DNCHUNK, DCH)

    deg = _deg_kernel(dst_d).T
    y1 = _prescale(deg, feat)
    g1 = _spmv_kernel(y1, src_p, dst_p)
    y2 = _mid(deg, g1)
    g2 = _spmv_kernel(y2, src_p, dst_p)
    return _final(deg, feat, y2, g2, W, bias.reshape(1, F))
